# Initial kernel scaffold; baseline (speedup 1.0000x reference)
import jax
import jax.numpy as jnp
from jax.experimental import pallas as pl


def kernel(x, edge_index, Wm0, bm0, Wv0, bv0, Wm1, bm1, Wv1, bv1, Wm2, bm2, Wv2, bv2):
    raise NotImplementedError("write your pallas kernel here")


# speedup vs baseline: 10.8849x; 10.8849x over previous
import jax
import jax.numpy as jnp
from jax import lax
from jax.experimental import pallas as pl
from jax.experimental.pallas import tpu as pltpu
from jax.experimental.pallas import tpu_sc as plsc

_N = 10000
_E = 320000
_F = 128
_NC = 2
_NS = 16
_K = 128
_NRP = 10240
_NDP = 10240
_EPAD = 323584
_BLK = 2000


def _sc_mesh():
    return plsc.VectorSubcoreMesh(
        core_axis_name="c", subcore_axis_name="s",
        num_cores=_NC, num_subcores=_NS)


def _deg_body(row_hbm, col_hbm, out_hbm,
              row_v, col_v, rowp_v, ones_v, seg_v, acc):
    c = lax.axis_index("c")
    s = lax.axis_index("s")
    seg = _NDP // _NS
    for i in range(seg // 16):
        seg_v[pl.ds(i * 16, 16)] = jnp.zeros((16,), jnp.float32)
    pltpu.sync_copy(seg_v, acc.at[pl.ds(s * seg, seg)])
    for i in range(_K // 16):
        ones_v[pl.ds(i * 16, 16)] = jnp.full((16,), 1.0, jnp.float32)
    plsc.subcore_barrier()
    chunks = _EPAD // (_NC * _NS * _K)
    base = (c * _NS + s) * chunks * _K

    def body(g, carry):
        off = base + g * _K
        pltpu.sync_copy(row_hbm.at[pl.ds(off, _K)], row_v)
        pltpu.sync_copy(col_hbm.at[pl.ds(off, _K)], col_v)
        for i in range(_K // 16):
            sl = pl.ds(i * 16, 16)
            r = row_v[sl]
            q = col_v[sl]
            rowp_v[sl] = jnp.where(r == q, _N, r)
        pltpu.sync_copy(ones_v, acc.at[rowp_v], add=True)
        return carry

    lax.fori_loop(0, chunks, body, 0)
    plsc.subcore_barrier()
    pltpu.sync_copy(acc.at[pl.ds(s * seg, seg)], seg_v)
    pltpu.sync_copy(seg_v, out_hbm.at[pl.ds(c * _NDP + s * seg, seg)])


def _spmm_body(src_hbm, row_hbm, col_hbm, out_hbm,
               row_v, col_v, rowp_v, col2_v, rows_v, acc, sem):
    c = lax.axis_index("c")
    s = lax.axis_index("s")
    rpt = _NRP // _NS
    coff = c * _NRP
    pltpu.sync_copy(src_hbm.at[pl.ds(coff + s * rpt, rpt)],
                    acc.at[pl.ds(s * rpt, rpt)])
    plsc.subcore_barrier()
    chunks = _EPAD // (_NS * _K)
    base = s * chunks * _K

    def body(g, carry):
        off = base + g * _K
        pltpu.sync_copy(row_hbm.at[pl.ds(off, _K)], row_v)
        pltpu.sync_copy(col_hbm.at[pl.ds(off, _K)], col_v)
        for i in range(_K // 16):
            sl = pl.ds(i * 16, 16)
            r = row_v[sl]
            q = col_v[sl]
            rowp_v[sl] = jnp.where(r == q, _N, r)
            col2_v[sl] = q + coff
        pltpu.async_copy(src_hbm.at[col2_v], rows_v, sem).wait()
        pltpu.sync_copy(rows_v, acc.at[rowp_v], add=True)
        return carry

    lax.fori_loop(0, chunks, body, 0)
    plsc.subcore_barrier()
    pltpu.sync_copy(acc.at[pl.ds(s * rpt, rpt)],
                    out_hbm.at[pl.ds(coff + s * rpt, rpt)])


_SC_CACHE = {}


def _deg_call(*args):
    if "deg" not in _SC_CACHE:
        _SC_CACHE["deg"] = pl.kernel(
            _deg_body,
            out_type=jax.ShapeDtypeStruct((_NC * _NDP,), jnp.float32),
            mesh=_sc_mesh(),
            scratch_types=[
                pltpu.VMEM((_K,), jnp.int32),
                pltpu.VMEM((_K,), jnp.int32),
                pltpu.VMEM((_K,), jnp.int32),
                pltpu.VMEM((_K,), jnp.float32),
                pltpu.VMEM((_NDP // _NS,), jnp.float32),
                pltpu.VMEM_SHARED((_NDP,), jnp.float32),
            ],
        )
    return _SC_CACHE["deg"](*args)


def _spmm_call(*args):
    if "spmm" not in _SC_CACHE:
        _SC_CACHE["spmm"] = pl.kernel(
            _spmm_body,
            out_type=jax.ShapeDtypeStruct((2 * _NRP, _F), jnp.float32),
            mesh=_sc_mesh(),
            scratch_types=[
                pltpu.VMEM((_K,), jnp.int32),
                pltpu.VMEM((_K,), jnp.int32),
                pltpu.VMEM((_K,), jnp.int32),
                pltpu.VMEM((_K,), jnp.int32),
                pltpu.VMEM((_K, _F), jnp.float32),
                pltpu.VMEM_SHARED((_NRP, _F), jnp.float32),
                pltpu.SemaphoreType.DMA,
            ],
        )
    return _SC_CACHE["spmm"](*args)


def _elu(t):
    return jnp.where(t > 0, t, jnp.exp(t) - 1.0)


def _dot(a, b):
    return jnp.dot(a, b, preferred_element_type=jnp.float32)


def _scales(da, db):
    deg = da[...] + db[...] + 1.0
    return lax.rsqrt(deg), 1.0 / deg


def _tc1_body(x, da, db, wm0, bm0, wv0, bv0, wm1, bm1, wv1, bv1, mo, vo):
    d0, d1 = _scales(da, db)
    xb = x[...]
    m = _elu(_dot(xb, wm0[...]) + bm0[...])
    v = jnp.maximum(_dot(xb, wv0[...]) + bv0[...], 0.0)
    m = _elu(_dot(m, wm1[...]) + bm1[...])
    v = jnp.maximum(_dot(v, wv1[...]) + bv1[...], 0.0) + 1e-6
    att = jnp.exp(-v)
    mo[...] = d0 * (m * att)
    vo[...] = d1 * (v * att * att)


def _tc2_body(ma, va, da, db, wm2, bm2, wv2, bv2, mo, vo):
    d0, d1 = _scales(da, db)
    m = d0 * ma[...]
    v = d1 * va[...]
    m = _elu(_dot(m, wm2[...]) + bm2[...])
    v = jnp.maximum(_dot(v, wv2[...]) + bv2[...], 0.0) + 1e-6
    att = jnp.exp(-v)
    mo[...] = d0 * (m * att)
    vo[...] = d1 * (v * att * att)


def _tc3_body(ma, va, da, db, smp, out):
    d0, d1 = _scales(da, db)
    m = d0 * ma[...]
    v = d1 * va[...]
    o = m + smp[...] * jnp.sqrt(v)
    o = o - jnp.max(o, axis=-1, keepdims=True)
    out[...] = o - jnp.log(jnp.sum(jnp.exp(o), axis=-1, keepdims=True))


def _row_spec():
    return pl.BlockSpec((_BLK, _F), lambda i: (i, 0))


def _deg_spec():
    return pl.BlockSpec((_BLK, 1), lambda i: (i, 0))


def _w_spec():
    return pl.BlockSpec((_F, _F), lambda i: (0, 0))


def _b_spec():
    return pl.BlockSpec((1, _F), lambda i: (0, 0))


_TC_PARAMS = pltpu.CompilerParams(dimension_semantics=("parallel",))


_tc1 = pl.pallas_call(
    _tc1_body,
    grid=(_N // _BLK,),
    in_specs=[_row_spec(), _deg_spec(), _deg_spec(),
              _w_spec(), _b_spec(), _w_spec(), _b_spec(),
              _w_spec(), _b_spec(), _w_spec(), _b_spec()],
    out_specs=[_row_spec(), _row_spec()],
    out_shape=[jax.ShapeDtypeStruct((_N, _F), jnp.float32)] * 2,
    compiler_params=_TC_PARAMS,
)

_tc2 = pl.pallas_call(
    _tc2_body,
    grid=(_N // _BLK,),
    in_specs=[_row_spec(), _row_spec(), _deg_spec(), _deg_spec(),
              _w_spec(), _b_spec(), _w_spec(), _b_spec()],
    out_specs=[_row_spec(), _row_spec()],
    out_shape=[jax.ShapeDtypeStruct((_N, _F), jnp.float32)] * 2,
    compiler_params=_TC_PARAMS,
)

_tc3 = pl.pallas_call(
    _tc3_body,
    grid=(_N // _BLK,),
    in_specs=[_row_spec(), _row_spec(), _deg_spec(), _deg_spec(), _row_spec()],
    out_specs=_row_spec(),
    out_shape=jax.ShapeDtypeStruct((_N, _F), jnp.float32),
    compiler_params=_TC_PARAMS,
)


def kernel(x, edge_index, Wm0, bm0, Wv0, bv0, Wm1, bm1, Wv1, bv1,
           Wm2, bm2, Wv2, bv2):
    n, f = x.shape
    row = edge_index[0]
    col = edge_index[1]
    e = row.shape[0]
    zpi = jnp.zeros((_EPAD - e,), jnp.int32)
    row_p = jnp.concatenate([row, zpi])
    col_p = jnp.concatenate([col, zpi])

    degp = _deg_call(row_p, col_p)
    da = degp[:n].reshape(n, 1)
    db = degp[_NDP:_NDP + n].reshape(n, 1)

    rb = lambda t: t.reshape(1, -1)
    mean_s, var_s = _tc1(x, da, db, Wm0, rb(bm0), Wv0, rb(bv0),
                         Wm1, rb(bm1), Wv1, rb(bv1))

    zrow = jnp.zeros((_NRP - n, f), jnp.float32)
    cat = jnp.concatenate([mean_s, zrow, var_s, zrow], axis=0)
    agg = _spmm_call(cat, row_p, col_p)
    ma, va = agg[:n], agg[_NRP:_NRP + n]

    mean_s2, var_s2 = _tc2(ma, va, da, db, Wm2, rb(bm2), Wv2, rb(bv2))
    cat2 = jnp.concatenate([mean_s2, zrow, var_s2, zrow], axis=0)
    agg2 = _spmm_call(cat2, row_p, col_p)
    ma2, va2 = agg2[:n], agg2[_NRP:_NRP + n]

    smp = jax.random.normal(jax.random.key(42), (n, f), jnp.float32)
    return _tc3(ma2, va2, da, db, smp)


# speedup vs baseline: 15.3005x; 1.4057x over previous
import jax
import jax.numpy as jnp
from jax import lax
from jax.experimental import pallas as pl
from jax.experimental.pallas import tpu as pltpu
from jax.experimental.pallas import tpu_sc as plsc

_N = 10000
_E = 320000
_F = 128
_NC = 2
_NS = 16
_K = 80
_NRP = 10240
_NDP = 10240
_EPAD = 322560
_CPT = _EPAD // (_NS * _K)
_NBUF = 4
_BLK = 2000


def _sc_mesh():
    return plsc.VectorSubcoreMesh(
        core_axis_name="c", subcore_axis_name="s",
        num_cores=_NC, num_subcores=_NS)


def _deg_body(row_hbm, col_hbm, out_hbm, rowp_hbm, colsh_hbm,
              row_v, col_v, rowp_v, colsh_v, ones_v, seg_v, acc):
    c = lax.axis_index("c")
    s = lax.axis_index("s")
    seg = _NDP // _NS
    for i in range(seg // 16):
        seg_v[pl.ds(i * 16, 16)] = jnp.zeros((16,), jnp.float32)
    pltpu.sync_copy(seg_v, acc.at[pl.ds(s * seg, seg)])
    for i in range(_K // 16):
        ones_v[pl.ds(i * 16, 16)] = jnp.full((16,), 1.0, jnp.float32)
    plsc.subcore_barrier()
    chunks = _EPAD // (_NC * _NS * _K)
    base = (c * _NS + s) * chunks * _K

    def body(g, carry):
        off = base + g * _K
        pltpu.sync_copy(row_hbm.at[pl.ds(off, _K)], row_v)
        pltpu.sync_copy(col_hbm.at[pl.ds(off, _K)], col_v)
        for i in range(_K // 16):
            sl = pl.ds(i * 16, 16)
            r = row_v[sl]
            q = col_v[sl]
            rowp_v[sl] = jnp.where(r == q, _N, r)
            colsh_v[sl] = q + _NRP
        pltpu.sync_copy(rowp_v, rowp_hbm.at[pl.ds(off, _K)])
        pltpu.sync_copy(colsh_v, colsh_hbm.at[pl.ds(off, _K)])
        pltpu.sync_copy(ones_v, acc.at[rowp_v], add=True)
        return carry

    lax.fori_loop(0, chunks, body, 0)
    plsc.subcore_barrier()
    pltpu.sync_copy(acc.at[pl.ds(s * seg, seg)], seg_v)
    pltpu.sync_copy(seg_v, out_hbm.at[pl.ds(c * _NDP + s * seg, seg)])


def _spmm_body(src_hbm, rowp_hbm, colcat_hbm, out_hbm,
               rp0, rp1, rp2, rp3, cl0, cl1, cl2, cl3,
               rows0, rows1, rows2, rows3, acc,
               g0, g1, g2, g3, i0, i1, i2, i3):
    c = lax.axis_index("c")
    s = lax.axis_index("s")
    rpt = _NRP // _NS
    rows = (rows0, rows1, rows2, rows3)
    rpc = (rp0, rp1, rp2, rp3)
    clc = (cl0, cl1, cl2, cl3)
    gsem = (g0, g1, g2, g3)
    isem = (i0, i1, i2, i3)
    pltpu.sync_copy(src_hbm.at[pl.ds(c * _NRP + s * rpt, rpt)],
                    acc.at[pl.ds(s * rpt, rpt)])
    plsc.subcore_barrier()

    base = s * _CPT * _K
    cbase = c * _EPAD + base

    def fire(slot, g):
        off = g * _K
        pltpu.async_copy(rowp_hbm.at[pl.ds(base + off, _K)], rpc[slot],
                         isem[slot])
        pltpu.async_copy(colcat_hbm.at[pl.ds(cbase + off, _K)], clc[slot],
                         isem[slot])
        pltpu.make_async_copy(rowp_hbm.at[pl.ds(0, _K)], rpc[slot],
                              isem[slot]).wait()
        pltpu.make_async_copy(rowp_hbm.at[pl.ds(0, _K)], clc[slot],
                              isem[slot]).wait()
        pltpu.async_copy(src_hbm.at[clc[slot]], rows[slot], gsem[slot])

    fire(0, 0)
    fire(1, 1)

    def body(p, carry):
        for u in range(_NBUF):
            g = p * _NBUF + u
            w = (u + 2) % _NBUF
            pltpu.make_async_copy(src_hbm.at[pl.ds(0, _K)],
                                  rows[u], gsem[u]).wait()
            pltpu.sync_copy(rows[u], acc.at[rpc[u]], add=True)

            @pl.when(g + 2 < _CPT)
            def _refill():
                fire(w, g + 2)
        return carry

    lax.fori_loop(0, _CPT // _NBUF, body, 0)
    plsc.subcore_barrier()
    pltpu.sync_copy(acc.at[pl.ds(s * rpt, rpt)],
                    out_hbm.at[pl.ds(c * _NRP + s * rpt, rpt)])


_SC_CACHE = {}


def _deg_call(*args):
    if "deg" not in _SC_CACHE:
        _SC_CACHE["deg"] = pl.kernel(
            _deg_body,
            out_type=[jax.ShapeDtypeStruct((_NC * _NDP,), jnp.float32),
                      jax.ShapeDtypeStruct((_EPAD,), jnp.int32),
                      jax.ShapeDtypeStruct((_EPAD,), jnp.int32)],
            mesh=_sc_mesh(),
            scratch_types=[
                pltpu.VMEM((_K,), jnp.int32),
                pltpu.VMEM((_K,), jnp.int32),
                pltpu.VMEM((_K,), jnp.int32),
                pltpu.VMEM((_K,), jnp.int32),
                pltpu.VMEM((_K,), jnp.float32),
                pltpu.VMEM((_NDP // _NS,), jnp.float32),
                pltpu.VMEM_SHARED((_NDP,), jnp.float32),
            ],
        )
    return _SC_CACHE["deg"](*args)


def _spmm_call(*args):
    if "spmm" not in _SC_CACHE:
        _SC_CACHE["spmm"] = pl.kernel(
            _spmm_body,
            out_type=jax.ShapeDtypeStruct((2 * _NRP, _F), jnp.float32),
            mesh=_sc_mesh(),
            scratch_types=(
                [pltpu.VMEM((_K,), jnp.int32)] * (2 * _NBUF)
                + [pltpu.VMEM((_K, _F), jnp.float32)] * _NBUF
                + [pltpu.VMEM_SHARED((_NRP, _F), jnp.float32)]
                + [pltpu.SemaphoreType.DMA] * (2 * _NBUF)
            ),
        )
    return _SC_CACHE["spmm"](*args)


def _elu(t):
    return jnp.where(t > 0, t, jnp.exp(t) - 1.0)


def _dot(a, b):
    return jnp.dot(a, b, preferred_element_type=jnp.float32)


def _scales(da, db):
    deg = da[...] + db[...] + 1.0
    return lax.rsqrt(deg), 1.0 / deg


def _tc1_body(x, da, db, wm0, bm0, wv0, bv0, wm1, bm1, wv1, bv1, mo, vo):
    d0, d1 = _scales(da, db)
    xb = x[...]
    m = _elu(_dot(xb, wm0[...]) + bm0[...])
    v = jnp.maximum(_dot(xb, wv0[...]) + bv0[...], 0.0)
    m = _elu(_dot(m, wm1[...]) + bm1[...])
    v = jnp.maximum(_dot(v, wv1[...]) + bv1[...], 0.0) + 1e-6
    att = jnp.exp(-v)
    mo[...] = d0 * (m * att)
    vo[...] = d1 * (v * att * att)


def _tc2_body(ma, va, da, db, wm2, bm2, wv2, bv2, mo, vo):
    d0, d1 = _scales(da, db)
    m = d0 * ma[...]
    v = d1 * va[...]
    m = _elu(_dot(m, wm2[...]) + bm2[...])
    v = jnp.maximum(_dot(v, wv2[...]) + bv2[...], 0.0) + 1e-6
    att = jnp.exp(-v)
    mo[...] = d0 * (m * att)
    vo[...] = d1 * (v * att * att)


def _tc3_body(ma, va, da, db, smp, out):
    d0, d1 = _scales(da, db)
    m = d0 * ma[...]
    v = d1 * va[...]
    o = m + smp[...] * jnp.sqrt(v)
    o = o - jnp.max(o, axis=-1, keepdims=True)
    out[...] = o - jnp.log(jnp.sum(jnp.exp(o), axis=-1, keepdims=True))


def _row_spec():
    return pl.BlockSpec((_BLK, _F), lambda i: (i, 0))


def _deg_spec():
    return pl.BlockSpec((_BLK, 1), lambda i: (i, 0))


def _w_spec():
    return pl.BlockSpec((_F, _F), lambda i: (0, 0))


def _b_spec():
    return pl.BlockSpec((1, _F), lambda i: (0, 0))


_TC_PARAMS = pltpu.CompilerParams(dimension_semantics=("parallel",))


_tc1 = pl.pallas_call(
    _tc1_body,
    grid=(_N // _BLK,),
    in_specs=[_row_spec(), _deg_spec(), _deg_spec(),
              _w_spec(), _b_spec(), _w_spec(), _b_spec(),
              _w_spec(), _b_spec(), _w_spec(), _b_spec()],
    out_specs=[_row_spec(), _row_spec()],
    out_shape=[jax.ShapeDtypeStruct((_N, _F), jnp.float32)] * 2,
    compiler_params=_TC_PARAMS,
)

_tc2 = pl.pallas_call(
    _tc2_body,
    grid=(_N // _BLK,),
    in_specs=[_row_spec(), _row_spec(), _deg_spec(), _deg_spec(),
              _w_spec(), _b_spec(), _w_spec(), _b_spec()],
    out_specs=[_row_spec(), _row_spec()],
    out_shape=[jax.ShapeDtypeStruct((_N, _F), jnp.float32)] * 2,
    compiler_params=_TC_PARAMS,
)

_tc3 = pl.pallas_call(
    _tc3_body,
    grid=(_N // _BLK,),
    in_specs=[_row_spec(), _row_spec(), _deg_spec(), _deg_spec(), _row_spec()],
    out_specs=_row_spec(),
    out_shape=jax.ShapeDtypeStruct((_N, _F), jnp.float32),
    compiler_params=_TC_PARAMS,
)


def kernel(x, edge_index, Wm0, bm0, Wv0, bv0, Wm1, bm1, Wv1, bv1,
           Wm2, bm2, Wv2, bv2):
    n, f = x.shape
    row = edge_index[0]
    col = edge_index[1]
    e = row.shape[0]
    zpi = jnp.zeros((_EPAD - e,), jnp.int32)
    row_p = jnp.concatenate([row, zpi])
    col_p = jnp.concatenate([col, zpi])

    degp, rowp, colsh = _deg_call(row_p, col_p)
    da = degp[:n].reshape(n, 1)
    db = degp[_NDP:_NDP + n].reshape(n, 1)
    colcat = jnp.concatenate([col_p, colsh])

    rb = lambda t: t.reshape(1, -1)
    mean_s, var_s = _tc1(x, da, db, Wm0, rb(bm0), Wv0, rb(bv0),
                         Wm1, rb(bm1), Wv1, rb(bv1))

    zrow = jnp.zeros((_NRP - n, f), jnp.float32)
    cat = jnp.concatenate([mean_s, zrow, var_s, zrow], axis=0)
    agg = _spmm_call(cat, rowp, colcat)
    ma, va = agg[:n], agg[_NRP:_NRP + n]

    mean_s2, var_s2 = _tc2(ma, va, da, db, Wm2, rb(bm2), Wv2, rb(bv2))
    cat2 = jnp.concatenate([mean_s2, zrow, var_s2, zrow], axis=0)
    agg2 = _spmm_call(cat2, rowp, colcat)
    ma2, va2 = agg2[:n], agg2[_NRP:_NRP + n]

    smp = jax.random.normal(jax.random.key(42), (n, f), jnp.float32)
    return _tc3(ma2, va2, da, db, smp)


# speedup vs baseline: 18.2555x; 1.1931x over previous
import jax
import jax.numpy as jnp
from jax import lax
from jax.experimental import pallas as pl
from jax.experimental.pallas import tpu as pltpu
from jax.experimental.pallas import tpu_sc as plsc

_N = 10000
_E = 320000
_F = 128
_NC = 2
_NS = 16
_K = 112
_NRP = 10112
_NDP = 10240
_EPAD = 322560
_CPT = _EPAD // (_NS * _K)
_NROWS = 3
_NIDX = 6
_BLK = 2000


def _sc_mesh():
    return plsc.VectorSubcoreMesh(
        core_axis_name="c", subcore_axis_name="s",
        num_cores=_NC, num_subcores=_NS)


def _deg_body(row_hbm, col_hbm, out_hbm, rowp_hbm, colsh_hbm,
              row_v, col_v, rowp_v, colsh_v, ones_v, seg_v, acc):
    c = lax.axis_index("c")
    s = lax.axis_index("s")
    seg = _NDP // _NS
    for i in range(seg // 16):
        seg_v[pl.ds(i * 16, 16)] = jnp.zeros((16,), jnp.float32)
    pltpu.sync_copy(seg_v, acc.at[pl.ds(s * seg, seg)])
    for i in range(_K // 16):
        ones_v[pl.ds(i * 16, 16)] = jnp.full((16,), 1.0, jnp.float32)
    plsc.subcore_barrier()
    chunks = _EPAD // (_NC * _NS * _K)
    base = (c * _NS + s) * chunks * _K

    def body(g, carry):
        off = base + g * _K
        pltpu.sync_copy(row_hbm.at[pl.ds(off, _K)], row_v)
        pltpu.sync_copy(col_hbm.at[pl.ds(off, _K)], col_v)
        for i in range(_K // 16):
            sl = pl.ds(i * 16, 16)
            r = row_v[sl]
            q = col_v[sl]
            rowp_v[sl] = jnp.where(r == q, _N, r)
            colsh_v[sl] = q + _NRP
        pltpu.sync_copy(rowp_v, rowp_hbm.at[pl.ds(off, _K)])
        pltpu.sync_copy(colsh_v, colsh_hbm.at[pl.ds(off, _K)])
        pltpu.sync_copy(ones_v, acc.at[rowp_v], add=True)
        return carry

    lax.fori_loop(0, chunks, body, 0)
    plsc.subcore_barrier()
    pltpu.sync_copy(acc.at[pl.ds(s * seg, seg)], seg_v)
    pltpu.sync_copy(seg_v, out_hbm.at[pl.ds(c * _NDP + s * seg, seg)])


def _spmm_body(src_hbm, rowp_hbm, colcat_hbm, out_hbm,
               rp0, rp1, rp2, rp3, rp4, rp5, cl0, cl1, cl2, cl3, cl4, cl5,
               rows0, rows1, rows2, acc,
               g0, g1, g2, i0, i1, i2, i3, i4, i5):
    c = lax.axis_index("c")
    s = lax.axis_index("s")
    rpt = _NRP // _NS
    rows = (rows0, rows1, rows2)
    rpc = (rp0, rp1, rp2, rp3, rp4, rp5)
    clc = (cl0, cl1, cl2, cl3, cl4, cl5)
    gsem = (g0, g1, g2)
    isem = (i0, i1, i2, i3, i4, i5)
    pltpu.sync_copy(src_hbm.at[pl.ds(c * _NRP + s * rpt, rpt)],
                    acc.at[pl.ds(s * rpt, rpt)])
    plsc.subcore_barrier()

    base = s * _CPT * _K
    cbase = c * _EPAD + base

    def fire_idx(q, g):
        off = g * _K
        pltpu.async_copy(rowp_hbm.at[pl.ds(base + off, _K)], rpc[q], isem[q])
        pltpu.async_copy(colcat_hbm.at[pl.ds(cbase + off, _K)], clc[q],
                         isem[q])

    def wait_idx(q):
        pltpu.make_async_copy(rowp_hbm.at[pl.ds(0, _K)], rpc[q],
                              isem[q]).wait()
        pltpu.make_async_copy(rowp_hbm.at[pl.ds(0, _K)], clc[q],
                              isem[q]).wait()

    def fire_gather(r, q):
        pltpu.async_copy(src_hbm.at[clc[q]], rows[r], gsem[r])

    for q in range(_NIDX):
        fire_idx(q, q)
    for g in range(2):
        wait_idx(g)
        fire_gather(g, g)

    def body(p, carry):
        for u in range(_NIDX):
            g = p * _NIDX + u
            r = u % _NROWS
            pltpu.make_async_copy(src_hbm.at[pl.ds(0, _K)],
                                  rows[r], gsem[r]).wait()
            pltpu.sync_copy(rows[r], acc.at[rpc[u]], add=True)

            @pl.when(g + _NIDX < _CPT)
            def _refill_idx():
                fire_idx(u, g + _NIDX)

            @pl.when(g + 2 < _CPT)
            def _next_gather():
                wait_idx((u + 2) % _NIDX)
                fire_gather((u + 2) % _NROWS, (u + 2) % _NIDX)
        return carry

    lax.fori_loop(0, _CPT // _NIDX, body, 0)
    plsc.subcore_barrier()
    pltpu.sync_copy(acc.at[pl.ds(s * rpt, rpt)],
                    out_hbm.at[pl.ds(c * _NRP + s * rpt, rpt)])


_SC_CACHE = {}


def _deg_call(*args):
    if "deg" not in _SC_CACHE:
        _SC_CACHE["deg"] = pl.kernel(
            _deg_body,
            out_type=[jax.ShapeDtypeStruct((_NC * _NDP,), jnp.float32),
                      jax.ShapeDtypeStruct((_EPAD,), jnp.int32),
                      jax.ShapeDtypeStruct((_EPAD,), jnp.int32)],
            mesh=_sc_mesh(),
            scratch_types=[
                pltpu.VMEM((_K,), jnp.int32),
                pltpu.VMEM((_K,), jnp.int32),
                pltpu.VMEM((_K,), jnp.int32),
                pltpu.VMEM((_K,), jnp.int32),
                pltpu.VMEM((_K,), jnp.float32),
                pltpu.VMEM((_NDP // _NS,), jnp.float32),
                pltpu.VMEM_SHARED((_NDP,), jnp.float32),
            ],
        )
    return _SC_CACHE["deg"](*args)


def _spmm_call(*args):
    if "spmm" not in _SC_CACHE:
        _SC_CACHE["spmm"] = pl.kernel(
            _spmm_body,
            out_type=jax.ShapeDtypeStruct((2 * _NRP, _F), jnp.float32),
            mesh=_sc_mesh(),
            scratch_types=(
                [pltpu.VMEM((_K,), jnp.int32)] * (2 * _NIDX)
                + [pltpu.VMEM((_K, _F), jnp.float32)] * _NROWS
                + [pltpu.VMEM_SHARED((_NRP, _F), jnp.float32)]
                + [pltpu.SemaphoreType.DMA] * (_NROWS + _NIDX)
            ),
        )
    return _SC_CACHE["spmm"](*args)


def _elu(t):
    return jnp.where(t > 0, t, jnp.exp(t) - 1.0)


def _dot(a, b):
    return jnp.dot(a, b, preferred_element_type=jnp.float32)


def _scales(da, db):
    deg = da[...] + db[...] + 1.0
    return lax.rsqrt(deg), 1.0 / deg


def _tc1_body(x, da, db, wm0, bm0, wv0, bv0, wm1, bm1, wv1, bv1, mo, vo):
    d0, d1 = _scales(da, db)
    xb = x[...]
    m = _elu(_dot(xb, wm0[...]) + bm0[...])
    v = jnp.maximum(_dot(xb, wv0[...]) + bv0[...], 0.0)
    m = _elu(_dot(m, wm1[...]) + bm1[...])
    v = jnp.maximum(_dot(v, wv1[...]) + bv1[...], 0.0) + 1e-6
    att = jnp.exp(-v)
    mo[...] = d0 * (m * att)
    vo[...] = d1 * (v * att * att)


def _tc2_body(ma, va, da, db, wm2, bm2, wv2, bv2, mo, vo):
    d0, d1 = _scales(da, db)
    m = d0 * ma[...]
    v = d1 * va[...]
    m = _elu(_dot(m, wm2[...]) + bm2[...])
    v = jnp.maximum(_dot(v, wv2[...]) + bv2[...], 0.0) + 1e-6
    att = jnp.exp(-v)
    mo[...] = d0 * (m * att)
    vo[...] = d1 * (v * att * att)


def _tc3_body(ma, va, da, db, smp, out):
    d0, d1 = _scales(da, db)
    m = d0 * ma[...]
    v = d1 * va[...]
    o = m + smp[...] * jnp.sqrt(v)
    o = o - jnp.max(o, axis=-1, keepdims=True)
    out[...] = o - jnp.log(jnp.sum(jnp.exp(o), axis=-1, keepdims=True))


def _row_spec():
    return pl.BlockSpec((_BLK, _F), lambda i: (i, 0))


def _deg_spec():
    return pl.BlockSpec((_BLK, 1), lambda i: (i, 0))


def _w_spec():
    return pl.BlockSpec((_F, _F), lambda i: (0, 0))


def _b_spec():
    return pl.BlockSpec((1, _F), lambda i: (0, 0))


_TC_PARAMS = pltpu.CompilerParams(dimension_semantics=("parallel",))


_tc1 = pl.pallas_call(
    _tc1_body,
    grid=(_N // _BLK,),
    in_specs=[_row_spec(), _deg_spec(), _deg_spec(),
              _w_spec(), _b_spec(), _w_spec(), _b_spec(),
              _w_spec(), _b_spec(), _w_spec(), _b_spec()],
    out_specs=[_row_spec(), _row_spec()],
    out_shape=[jax.ShapeDtypeStruct((_N, _F), jnp.float32)] * 2,
    compiler_params=_TC_PARAMS,
)

_tc2 = pl.pallas_call(
    _tc2_body,
    grid=(_N // _BLK,),
    in_specs=[_row_spec(), _row_spec(), _deg_spec(), _deg_spec(),
              _w_spec(), _b_spec(), _w_spec(), _b_spec()],
    out_specs=[_row_spec(), _row_spec()],
    out_shape=[jax.ShapeDtypeStruct((_N, _F), jnp.float32)] * 2,
    compiler_params=_TC_PARAMS,
)

_tc3 = pl.pallas_call(
    _tc3_body,
    grid=(_N // _BLK,),
    in_specs=[_row_spec(), _row_spec(), _deg_spec(), _deg_spec(), _row_spec()],
    out_specs=_row_spec(),
    out_shape=jax.ShapeDtypeStruct((_N, _F), jnp.float32),
    compiler_params=_TC_PARAMS,
)


def kernel(x, edge_index, Wm0, bm0, Wv0, bv0, Wm1, bm1, Wv1, bv1,
           Wm2, bm2, Wv2, bv2):
    n, f = x.shape
    row = edge_index[0]
    col = edge_index[1]
    e = row.shape[0]
    zpi = jnp.zeros((_EPAD - e,), jnp.int32)
    row_p = jnp.concatenate([row, zpi])
    col_p = jnp.concatenate([col, zpi])

    degp, rowp, colsh = _deg_call(row_p, col_p)
    da = degp[:n].reshape(n, 1)
    db = degp[_NDP:_NDP + n].reshape(n, 1)
    colcat = jnp.concatenate([col_p, colsh])

    rb = lambda t: t.reshape(1, -1)
    mean_s, var_s = _tc1(x, da, db, Wm0, rb(bm0), Wv0, rb(bv0),
                         Wm1, rb(bm1), Wv1, rb(bv1))

    zrow = jnp.zeros((_NRP - n, f), jnp.float32)
    cat = jnp.concatenate([mean_s, zrow, var_s, zrow], axis=0)
    agg = _spmm_call(cat, rowp, colcat)
    ma, va = agg[:n], agg[_NRP:_NRP + n]

    mean_s2, var_s2 = _tc2(ma, va, da, db, Wm2, rb(bm2), Wv2, rb(bv2))
    cat2 = jnp.concatenate([mean_s2, zrow, var_s2, zrow], axis=0)
    agg2 = _spmm_call(cat2, rowp, colcat)
    ma2, va2 = agg2[:n], agg2[_NRP:_NRP + n]

    smp = jax.random.normal(jax.random.key(42), (n, f), jnp.float32)
    return _tc3(ma2, va2, da, db, smp)


# speedup vs baseline: 19.7786x; 1.0834x over previous
import jax
import jax.numpy as jnp
from jax import lax
from jax.experimental import pallas as pl
from jax.experimental.pallas import tpu as pltpu
from jax.experimental.pallas import tpu_sc as plsc

_N = 10000
_E = 320000
_F = 128
_NC = 2
_NS = 16
_K = 120
_KD = 80
_NRP = 10112
_NDP = 10240
_EPAD = 322560
_CPT = _EPAD // (_NS * _K)
_NROWS = 3
_NIDX = 6
_BLK = 2000


def _sc_mesh():
    return plsc.VectorSubcoreMesh(
        core_axis_name="c", subcore_axis_name="s",
        num_cores=_NC, num_subcores=_NS)


def _deg_body(row_hbm, col_hbm, out_hbm, rowp_hbm, colsh_hbm,
              r0, r1, c0, c1, p0, p1, h0, h1, ones_v, seg_v, acc,
              l0, l1, w0, w1):
    c = lax.axis_index("c")
    s = lax.axis_index("s")
    seg = _NDP // _NS
    row_v = (r0, r1)
    col_v = (c0, c1)
    rowp_v = (p0, p1)
    colsh_v = (h0, h1)
    lsem = (l0, l1)
    wsem = (w0, w1)
    for i in range(seg // 16):
        seg_v[pl.ds(i * 16, 16)] = jnp.zeros((16,), jnp.float32)
    pltpu.sync_copy(seg_v, acc.at[pl.ds(s * seg, seg)])
    for i in range(_KD // 16):
        ones_v[pl.ds(i * 16, 16)] = jnp.full((16,), 1.0, jnp.float32)
    plsc.subcore_barrier()
    chunks = _EPAD // (_NC * _NS * _KD)
    base = (c * _NS + s) * chunks * _KD

    def load(b, g):
        off = base + g * _KD
        pltpu.async_copy(row_hbm.at[pl.ds(off, _KD)], row_v[b], lsem[b])
        pltpu.async_copy(col_hbm.at[pl.ds(off, _KD)], col_v[b], lsem[b])

    load(0, 0)
    load(1, 1)

    def body(p, carry):
        for b in range(2):
            g = p * 2 + b

            @pl.when(g >= 2)
            def _wdrain():
                pltpu.make_async_copy(row_hbm.at[pl.ds(0, _KD)],
                                      rowp_v[b], wsem[b]).wait()
                pltpu.make_async_copy(row_hbm.at[pl.ds(0, _KD)],
                                      colsh_v[b], wsem[b]).wait()

            pltpu.make_async_copy(row_hbm.at[pl.ds(0, _KD)],
                                  row_v[b], lsem[b]).wait()
            pltpu.make_async_copy(row_hbm.at[pl.ds(0, _KD)],
                                  col_v[b], lsem[b]).wait()
            for i in range(_KD // 16):
                sl = pl.ds(i * 16, 16)
                r = row_v[b][sl]
                q = col_v[b][sl]
                rowp_v[b][sl] = jnp.where(r == q, _N, r)
                colsh_v[b][sl] = q + _NRP
            off = base + g * _KD
            pltpu.async_copy(rowp_v[b], rowp_hbm.at[pl.ds(off, _KD)], wsem[b])
            pltpu.async_copy(colsh_v[b], colsh_hbm.at[pl.ds(off, _KD)],
                             wsem[b])
            pltpu.sync_copy(ones_v, acc.at[rowp_v[b]], add=True)

            @pl.when(g + 2 < chunks)
            def _refill():
                load(b, g + 2)
        return carry

    lax.fori_loop(0, chunks // 2, body, 0)
    for b in range(2):
        pltpu.make_async_copy(row_hbm.at[pl.ds(0, _KD)],
                              rowp_v[b], wsem[b]).wait()
        pltpu.make_async_copy(row_hbm.at[pl.ds(0, _KD)],
                              colsh_v[b], wsem[b]).wait()
    plsc.subcore_barrier()
    pltpu.sync_copy(acc.at[pl.ds(s * seg, seg)], seg_v)
    pltpu.sync_copy(seg_v, out_hbm.at[pl.ds(c * _NDP + s * seg, seg)])


def _spmm_body(src_hbm, rowp_hbm, colcat_hbm, out_hbm,
               rp0, rp1, rp2, rp3, rp4, rp5, cl0, cl1, cl2, cl3, cl4, cl5,
               rows0, rows1, rows2, acc,
               g0, g1, g2, i0, i1, i2, i3, i4, i5):
    c = lax.axis_index("c")
    s = lax.axis_index("s")
    rpt = _NRP // _NS
    rows = (rows0, rows1, rows2)
    rpc = (rp0, rp1, rp2, rp3, rp4, rp5)
    clc = (cl0, cl1, cl2, cl3, cl4, cl5)
    gsem = (g0, g1, g2)
    isem = (i0, i1, i2, i3, i4, i5)
    pltpu.sync_copy(src_hbm.at[pl.ds(c * _NRP + s * rpt, rpt)],
                    acc.at[pl.ds(s * rpt, rpt)])
    plsc.subcore_barrier()

    base = s * _CPT * _K
    cbase = c * _EPAD + base

    def fire_idx(q, g):
        off = g * _K
        pltpu.async_copy(rowp_hbm.at[pl.ds(base + off, _K)], rpc[q], isem[q])
        pltpu.async_copy(colcat_hbm.at[pl.ds(cbase + off, _K)], clc[q],
                         isem[q])

    def wait_idx(q):
        pltpu.make_async_copy(rowp_hbm.at[pl.ds(0, _K)], rpc[q],
                              isem[q]).wait()
        pltpu.make_async_copy(rowp_hbm.at[pl.ds(0, _K)], clc[q],
                              isem[q]).wait()

    def fire_gather(r, q):
        pltpu.async_copy(src_hbm.at[clc[q]], rows[r], gsem[r])

    for q in range(_NIDX):
        fire_idx(q, q)
    for g in range(2):
        wait_idx(g)
        fire_gather(g, g)

    def body(p, carry):
        for u in range(_NIDX):
            g = p * _NIDX + u
            r = u % _NROWS
            pltpu.make_async_copy(src_hbm.at[pl.ds(0, _K)],
                                  rows[r], gsem[r]).wait()
            pltpu.sync_copy(rows[r], acc.at[rpc[u]], add=True)

            @pl.when(g + _NIDX < _CPT)
            def _refill_idx():
                fire_idx(u, g + _NIDX)

            @pl.when(g + 2 < _CPT)
            def _next_gather():
                wait_idx((u + 2) % _NIDX)
                fire_gather((u + 2) % _NROWS, (u + 2) % _NIDX)
        return carry

    lax.fori_loop(0, _CPT // _NIDX, body, 0)
    plsc.subcore_barrier()
    pltpu.sync_copy(acc.at[pl.ds(s * rpt, rpt)],
                    out_hbm.at[pl.ds(c * _NRP + s * rpt, rpt)])


_SC_CACHE = {}


def _deg_call(*args):
    if "deg" not in _SC_CACHE:
        _SC_CACHE["deg"] = pl.kernel(
            _deg_body,
            out_type=[jax.ShapeDtypeStruct((_NC * _NDP,), jnp.float32),
                      jax.ShapeDtypeStruct((_EPAD,), jnp.int32),
                      jax.ShapeDtypeStruct((_EPAD,), jnp.int32)],
            mesh=_sc_mesh(),
            scratch_types=(
                [pltpu.VMEM((_KD,), jnp.int32)] * 8
                + [pltpu.VMEM((_KD,), jnp.float32),
                   pltpu.VMEM((_NDP // _NS,), jnp.float32),
                   pltpu.VMEM_SHARED((_NDP,), jnp.float32)]
                + [pltpu.SemaphoreType.DMA] * 4
            ),
        )
    return _SC_CACHE["deg"](*args)


def _spmm_call(*args):
    if "spmm" not in _SC_CACHE:
        _SC_CACHE["spmm"] = pl.kernel(
            _spmm_body,
            out_type=jax.ShapeDtypeStruct((2 * _NRP, _F), jnp.float32),
            mesh=_sc_mesh(),
            scratch_types=(
                [pltpu.VMEM((_K,), jnp.int32)] * (2 * _NIDX)
                + [pltpu.VMEM((_K, _F), jnp.float32)] * _NROWS
                + [pltpu.VMEM_SHARED((_NRP, _F), jnp.float32)]
                + [pltpu.SemaphoreType.DMA] * (_NROWS + _NIDX)
            ),
        )
    return _SC_CACHE["spmm"](*args)


def _elu(t):
    return jnp.where(t > 0, t, jnp.exp(t) - 1.0)


def _dot(a, b):
    return jnp.dot(a, b, preferred_element_type=jnp.float32)


def _scales(da, db):
    deg = da[...] + db[...] + 1.0
    return lax.rsqrt(deg), 1.0 / deg


def _tc1_body(x, da, db, wm0, bm0, wv0, bv0, wm1, bm1, wv1, bv1, mo, vo):
    d0, d1 = _scales(da, db)
    xb = x[...]
    m = _elu(_dot(xb, wm0[...]) + bm0[...])
    v = jnp.maximum(_dot(xb, wv0[...]) + bv0[...], 0.0)
    m = _elu(_dot(m, wm1[...]) + bm1[...])
    v = jnp.maximum(_dot(v, wv1[...]) + bv1[...], 0.0) + 1e-6
    att = jnp.exp(-v)
    mo[...] = d0 * (m * att)
    vo[...] = d1 * (v * att * att)


def _tc2_body(ma, va, da, db, wm2, bm2, wv2, bv2, mo, vo):
    d0, d1 = _scales(da, db)
    m = d0 * ma[...]
    v = d1 * va[...]
    m = _elu(_dot(m, wm2[...]) + bm2[...])
    v = jnp.maximum(_dot(v, wv2[...]) + bv2[...], 0.0) + 1e-6
    att = jnp.exp(-v)
    mo[...] = d0 * (m * att)
    vo[...] = d1 * (v * att * att)


def _tc3_body(ma, va, da, db, smp, out):
    d0, d1 = _scales(da, db)
    m = d0 * ma[...]
    v = d1 * va[...]
    o = m + smp[...] * jnp.sqrt(v)
    o = o - jnp.max(o, axis=-1, keepdims=True)
    out[...] = o - jnp.log(jnp.sum(jnp.exp(o), axis=-1, keepdims=True))


def _row_spec():
    return pl.BlockSpec((_BLK, _F), lambda i: (i, 0))


def _deg_spec():
    return pl.BlockSpec((_BLK, 1), lambda i: (i, 0))


def _w_spec():
    return pl.BlockSpec((_F, _F), lambda i: (0, 0))


def _b_spec():
    return pl.BlockSpec((1, _F), lambda i: (0, 0))


_TC_PARAMS = pltpu.CompilerParams(dimension_semantics=("parallel",))


_tc1 = pl.pallas_call(
    _tc1_body,
    grid=(_N // _BLK,),
    in_specs=[_row_spec(), _deg_spec(), _deg_spec(),
              _w_spec(), _b_spec(), _w_spec(), _b_spec(),
              _w_spec(), _b_spec(), _w_spec(), _b_spec()],
    out_specs=[_row_spec(), _row_spec()],
    out_shape=[jax.ShapeDtypeStruct((_N, _F), jnp.float32)] * 2,
    compiler_params=_TC_PARAMS,
)

_tc2 = pl.pallas_call(
    _tc2_body,
    grid=(_N // _BLK,),
    in_specs=[_row_spec(), _row_spec(), _deg_spec(), _deg_spec(),
              _w_spec(), _b_spec(), _w_spec(), _b_spec()],
    out_specs=[_row_spec(), _row_spec()],
    out_shape=[jax.ShapeDtypeStruct((_N, _F), jnp.float32)] * 2,
    compiler_params=_TC_PARAMS,
)

_tc3 = pl.pallas_call(
    _tc3_body,
    grid=(_N // _BLK,),
    in_specs=[_row_spec(), _row_spec(), _deg_spec(), _deg_spec(), _row_spec()],
    out_specs=_row_spec(),
    out_shape=jax.ShapeDtypeStruct((_N, _F), jnp.float32),
    compiler_params=_TC_PARAMS,
)


def kernel(x, edge_index, Wm0, bm0, Wv0, bv0, Wm1, bm1, Wv1, bv1,
           Wm2, bm2, Wv2, bv2):
    n, f = x.shape
    row = edge_index[0]
    col = edge_index[1]
    e = row.shape[0]
    zpi = jnp.zeros((_EPAD - e,), jnp.int32)
    row_p = jnp.concatenate([row, zpi])
    col_p = jnp.concatenate([col, zpi])

    degp, rowp, colsh = _deg_call(row_p, col_p)
    da = degp[:n].reshape(n, 1)
    db = degp[_NDP:_NDP + n].reshape(n, 1)
    colcat = jnp.concatenate([col_p, colsh])

    rb = lambda t: t.reshape(1, -1)
    mean_s, var_s = _tc1(x, da, db, Wm0, rb(bm0), Wv0, rb(bv0),
                         Wm1, rb(bm1), Wv1, rb(bv1))

    zrow = jnp.zeros((_NRP - n, f), jnp.float32)
    cat = jnp.concatenate([mean_s, zrow, var_s, zrow], axis=0)
    agg = _spmm_call(cat, rowp, colcat)
    ma, va = agg[:n], agg[_NRP:_NRP + n]

    mean_s2, var_s2 = _tc2(ma, va, da, db, Wm2, rb(bm2), Wv2, rb(bv2))
    cat2 = jnp.concatenate([mean_s2, zrow, var_s2, zrow], axis=0)
    agg2 = _spmm_call(cat2, rowp, colcat)
    ma2, va2 = agg2[:n], agg2[_NRP:_NRP + n]

    smp = jax.random.normal(jax.random.key(42), (n, f), jnp.float32)
    return _tc3(ma2, va2, da, db, smp)


# speedup vs baseline: 22.5717x; 1.1412x over previous
import jax
import jax.numpy as jnp
from jax import lax
from jax.experimental import pallas as pl
from jax.experimental.pallas import tpu as pltpu
from jax.experimental.pallas import tpu_sc as plsc

_N = 10000
_E = 320000
_F = 128
_NC = 2
_NS = 16
_K = 120
_KD = 80
_NRP = 10112
_NDP = 10240
_EPAD = 322560
_CPT = _EPAD // (_NS * _K)
_NROWS = 3
_NIDX = 6
_BLK = 2000


def _sc_mesh():
    return plsc.VectorSubcoreMesh(
        core_axis_name="c", subcore_axis_name="s",
        num_cores=_NC, num_subcores=_NS)


def _deg_body(row_hbm, col_hbm, out_hbm, rowp_hbm,
              r0, r1, c0, c1, p0, p1, ones_v, seg_v, acc,
              l0, l1, w0, w1):
    c = lax.axis_index("c")
    s = lax.axis_index("s")
    seg = _NDP // _NS
    row_v = (r0, r1)
    col_v = (c0, c1)
    rowp_v = (p0, p1)
    lsem = (l0, l1)
    wsem = (w0, w1)
    for i in range(seg // 16):
        seg_v[pl.ds(i * 16, 16)] = jnp.zeros((16,), jnp.float32)
    pltpu.sync_copy(seg_v, acc.at[pl.ds(s * seg, seg)])
    for i in range(_KD // 16):
        ones_v[pl.ds(i * 16, 16)] = jnp.full((16,), 1.0, jnp.float32)
    plsc.subcore_barrier()
    chunks = _EPAD // (_NC * _NS * _KD)
    base = (c * _NS + s) * chunks * _KD

    def load(b, g):
        off = base + g * _KD
        pltpu.async_copy(row_hbm.at[pl.ds(off, _KD)], row_v[b], lsem[b])
        pltpu.async_copy(col_hbm.at[pl.ds(off, _KD)], col_v[b], lsem[b])

    load(0, 0)
    load(1, 1)

    def body(p, carry):
        for b in range(2):
            g = p * 2 + b

            @pl.when(g >= 2)
            def _wdrain():
                pltpu.make_async_copy(row_hbm.at[pl.ds(0, _KD)],
                                      rowp_v[b], wsem[b]).wait()

            pltpu.make_async_copy(row_hbm.at[pl.ds(0, _KD)],
                                  row_v[b], lsem[b]).wait()
            pltpu.make_async_copy(row_hbm.at[pl.ds(0, _KD)],
                                  col_v[b], lsem[b]).wait()
            for i in range(_KD // 16):
                sl = pl.ds(i * 16, 16)
                r = row_v[b][sl]
                q = col_v[b][sl]
                rowp_v[b][sl] = jnp.where(r == q, _N, r)
            off = base + g * _KD
            pltpu.async_copy(rowp_v[b], rowp_hbm.at[pl.ds(off, _KD)], wsem[b])
            pltpu.sync_copy(ones_v, acc.at[rowp_v[b]], add=True)

            @pl.when(g + 2 < chunks)
            def _refill():
                load(b, g + 2)
        return carry

    lax.fori_loop(0, chunks // 2, body, 0)
    for b in range(2):
        pltpu.make_async_copy(row_hbm.at[pl.ds(0, _KD)],
                              rowp_v[b], wsem[b]).wait()
    plsc.subcore_barrier()
    pltpu.sync_copy(acc.at[pl.ds(s * seg, seg)], seg_v)
    pltpu.sync_copy(seg_v, out_hbm.at[pl.ds(c * _NDP + s * seg, seg)])


def _spmm_body(mean_hbm, var_hbm, rowp_hbm, col_hbm, mo_hbm, vo_hbm,
               rp0, rp1, rp2, rp3, rp4, rp5, cl0, cl1, cl2, cl3, cl4, cl5,
               rows0, rows1, rows2, acc,
               g0, g1, g2, i0, i1, i2, i3, i4, i5):
    c = lax.axis_index("c")
    s = lax.axis_index("s")
    rpt = _NRP // _NS
    rows = (rows0, rows1, rows2)
    rpc = (rp0, rp1, rp2, rp3, rp4, rp5)
    clc = (cl0, cl1, cl2, cl3, cl4, cl5)
    gsem = (g0, g1, g2)
    isem = (i0, i1, i2, i3, i4, i5)
    base = s * _CPT * _K

    def fire_idx(q, g):
        off = base + g * _K
        pltpu.async_copy(rowp_hbm.at[pl.ds(off, _K)], rpc[q], isem[q])
        pltpu.async_copy(col_hbm.at[pl.ds(off, _K)], clc[q], isem[q])

    def wait_idx(q):
        pltpu.make_async_copy(rowp_hbm.at[pl.ds(0, _K)], rpc[q],
                              isem[q]).wait()
        pltpu.make_async_copy(rowp_hbm.at[pl.ds(0, _K)], clc[q],
                              isem[q]).wait()

    def run(src_hbm, out_hbm):
        pltpu.sync_copy(src_hbm.at[pl.ds(s * rpt, rpt)],
                        acc.at[pl.ds(s * rpt, rpt)])
        plsc.subcore_barrier()

        def fire_gather(r, q):
            pltpu.async_copy(src_hbm.at[clc[q]], rows[r], gsem[r])

        for q in range(_NIDX):
            fire_idx(q, q)
        for g in range(2):
            wait_idx(g)
            fire_gather(g, g)

        def body(p, carry):
            for u in range(_NIDX):
                g = p * _NIDX + u
                r = u % _NROWS
                pltpu.make_async_copy(src_hbm.at[pl.ds(0, _K)],
                                      rows[r], gsem[r]).wait()
                pltpu.sync_copy(rows[r], acc.at[rpc[u]], add=True)

                @pl.when(g + _NIDX < _CPT)
                def _refill_idx():
                    fire_idx(u, g + _NIDX)

                @pl.when(g + 2 < _CPT)
                def _next_gather():
                    wait_idx((u + 2) % _NIDX)
                    fire_gather((u + 2) % _NROWS, (u + 2) % _NIDX)
            return carry

        lax.fori_loop(0, _CPT // _NIDX, body, 0)
        plsc.subcore_barrier()
        pltpu.sync_copy(acc.at[pl.ds(s * rpt, rpt)],
                        out_hbm.at[pl.ds(s * rpt, rpt)])

    @pl.when(c == 0)
    def _mean():
        run(mean_hbm, mo_hbm)

    @pl.when(c == 1)
    def _var():
        run(var_hbm, vo_hbm)


_SC_CACHE = {}


def _deg_call(*args):
    if "deg" not in _SC_CACHE:
        _SC_CACHE["deg"] = pl.kernel(
            _deg_body,
            out_type=[jax.ShapeDtypeStruct((_NC * _NDP,), jnp.float32),
                      jax.ShapeDtypeStruct((_EPAD,), jnp.int32)],
            mesh=_sc_mesh(),
            scratch_types=(
                [pltpu.VMEM((_KD,), jnp.int32)] * 6
                + [pltpu.VMEM((_KD,), jnp.float32),
                   pltpu.VMEM((_NDP // _NS,), jnp.float32),
                   pltpu.VMEM_SHARED((_NDP,), jnp.float32)]
                + [pltpu.SemaphoreType.DMA] * 4
            ),
        )
    return _SC_CACHE["deg"](*args)


def _spmm_call(*args):
    if "spmm" not in _SC_CACHE:
        _SC_CACHE["spmm"] = pl.kernel(
            _spmm_body,
            out_type=[jax.ShapeDtypeStruct((_NRP, _F), jnp.float32)] * 2,
            mesh=_sc_mesh(),
            scratch_types=(
                [pltpu.VMEM((_K,), jnp.int32)] * (2 * _NIDX)
                + [pltpu.VMEM((_K, _F), jnp.float32)] * _NROWS
                + [pltpu.VMEM_SHARED((_NRP, _F), jnp.float32)]
                + [pltpu.SemaphoreType.DMA] * (_NROWS + _NIDX)
            ),
        )
    return _SC_CACHE["spmm"](*args)


def _elu(t):
    return jnp.where(t > 0, t, jnp.exp(t) - 1.0)


def _dot(a, b):
    return jnp.dot(a, b, preferred_element_type=jnp.float32)


def _scales(da, db):
    deg = da[...] + db[...] + 1.0
    return lax.rsqrt(deg), 1.0 / deg


def _tc1a_body(x, wm0, bm0, wv0, bv0, wm1, bm1, wv1, bv1, mo, vo):
    xb = x[...]
    m = _elu(_dot(xb, wm0[...]) + bm0[...])
    v = jnp.maximum(_dot(xb, wv0[...]) + bv0[...], 0.0)
    m = _elu(_dot(m, wm1[...]) + bm1[...])
    v = jnp.maximum(_dot(v, wv1[...]) + bv1[...], 0.0) + 1e-6
    att = jnp.exp(-v)
    mo[...] = m * att
    vo[...] = v * att * att


def _tc1b_body(m, v, da, db, mo, vo):
    d0, d1 = _scales(da, db)
    mo[...] = d0 * m[...]
    vo[...] = d1 * v[...]


def _tc2_body(ma, va, da, db, wm2, bm2, wv2, bv2, mo, vo):
    d0, d1 = _scales(da, db)
    m = d0 * ma[...]
    v = d1 * va[...]
    m = _elu(_dot(m, wm2[...]) + bm2[...])
    v = jnp.maximum(_dot(v, wv2[...]) + bv2[...], 0.0) + 1e-6
    att = jnp.exp(-v)
    mo[...] = d0 * (m * att)
    vo[...] = d1 * (v * att * att)


def _tc3_body(ma, va, da, db, smp, out):
    d0, d1 = _scales(da, db)
    m = d0 * ma[...]
    v = d1 * va[...]
    o = m + smp[...] * jnp.sqrt(v)
    o = o - jnp.max(o, axis=-1, keepdims=True)
    out[...] = o - jnp.log(jnp.sum(jnp.exp(o), axis=-1, keepdims=True))


def _row_spec():
    return pl.BlockSpec((_BLK, _F), lambda i: (i, 0))


def _deg_spec():
    return pl.BlockSpec((_BLK, 1), lambda i: (i, 0))


def _w_spec():
    return pl.BlockSpec((_F, _F), lambda i: (0, 0))


def _b_spec():
    return pl.BlockSpec((1, _F), lambda i: (0, 0))


_TC_PARAMS = pltpu.CompilerParams(dimension_semantics=("parallel",))

_PADDED_OUT = [jax.ShapeDtypeStruct((_NRP, _F), jnp.float32)] * 2

_tc1a = pl.pallas_call(
    _tc1a_body,
    grid=(_N // _BLK,),
    in_specs=[_row_spec(),
              _w_spec(), _b_spec(), _w_spec(), _b_spec(),
              _w_spec(), _b_spec(), _w_spec(), _b_spec()],
    out_specs=[_row_spec(), _row_spec()],
    out_shape=[jax.ShapeDtypeStruct((_N, _F), jnp.float32)] * 2,
    compiler_params=_TC_PARAMS,
)

_tc1b = pl.pallas_call(
    _tc1b_body,
    grid=(_N // _BLK,),
    in_specs=[_row_spec(), _row_spec(), _deg_spec(), _deg_spec()],
    out_specs=[_row_spec(), _row_spec()],
    out_shape=_PADDED_OUT,
    compiler_params=_TC_PARAMS,
)

_tc2 = pl.pallas_call(
    _tc2_body,
    grid=(_N // _BLK,),
    in_specs=[_row_spec(), _row_spec(), _deg_spec(), _deg_spec(),
              _w_spec(), _b_spec(), _w_spec(), _b_spec()],
    out_specs=[_row_spec(), _row_spec()],
    out_shape=_PADDED_OUT,
    compiler_params=_TC_PARAMS,
)

_tc3 = pl.pallas_call(
    _tc3_body,
    grid=(_N // _BLK,),
    in_specs=[_row_spec(), _row_spec(), _deg_spec(), _deg_spec(), _row_spec()],
    out_specs=_row_spec(),
    out_shape=jax.ShapeDtypeStruct((_N, _F), jnp.float32),
    compiler_params=_TC_PARAMS,
)


def kernel(x, edge_index, Wm0, bm0, Wv0, bv0, Wm1, bm1, Wv1, bv1,
           Wm2, bm2, Wv2, bv2):
    n, f = x.shape
    row = edge_index[0]
    col = edge_index[1]
    e = row.shape[0]
    zpi = jnp.zeros((_EPAD - e,), jnp.int32)
    row_p = jnp.concatenate([row, zpi])
    col_p = jnp.concatenate([col, zpi])

    degp, rowp = _deg_call(row_p, col_p)
    da = degp[:n].reshape(n, 1)
    db = degp[_NDP:_NDP + n].reshape(n, 1)

    rb = lambda t: t.reshape(1, -1)
    m1, v1 = _tc1a(x, Wm0, rb(bm0), Wv0, rb(bv0),
                   Wm1, rb(bm1), Wv1, rb(bv1))
    mean_s, var_s = _tc1b(m1, v1, da, db)

    ma, va = _spmm_call(mean_s, var_s, rowp, col_p)
    mean_s2, var_s2 = _tc2(ma, va, da, db, Wm2, rb(bm2), Wv2, rb(bv2))
    ma2, va2 = _spmm_call(mean_s2, var_s2, rowp, col_p)

    smp = jax.random.normal(jax.random.key(42), (n, f), jnp.float32)
    return _tc3(ma2, va2, da, db, smp)


# speedup vs baseline: 23.5702x; 1.0442x over previous
import jax
import jax.numpy as jnp
from jax import lax
from jax.experimental import pallas as pl
from jax.experimental.pallas import tpu as pltpu
from jax.experimental.pallas import tpu_sc as plsc

_N = 10000
_E = 320000
_F = 128
_NC = 2
_NS = 16
_K = 80
_KD = 80
_NRP = 10112
_NDP = 10240
_EPAD = 322560
_CPT = _EPAD // (_NS * _K)
_NROWS = 4
_NIDX = 6
_UNROLL = 12
_BLK = 2000


def _sc_mesh():
    return plsc.VectorSubcoreMesh(
        core_axis_name="c", subcore_axis_name="s",
        num_cores=_NC, num_subcores=_NS)


def _deg_body(row_hbm, col_hbm, out_hbm, rowp_hbm,
              r0, r1, c0, c1, p0, p1, ones_v, seg_v, acc,
              l0, l1, w0, w1):
    c = lax.axis_index("c")
    s = lax.axis_index("s")
    seg = _NDP // _NS
    row_v = (r0, r1)
    col_v = (c0, c1)
    rowp_v = (p0, p1)
    lsem = (l0, l1)
    wsem = (w0, w1)
    for i in range(seg // 16):
        seg_v[pl.ds(i * 16, 16)] = jnp.zeros((16,), jnp.float32)
    pltpu.sync_copy(seg_v, acc.at[pl.ds(s * seg, seg)])
    for i in range(_KD // 16):
        ones_v[pl.ds(i * 16, 16)] = jnp.full((16,), 1.0, jnp.float32)
    plsc.subcore_barrier()
    chunks = _EPAD // (_NC * _NS * _KD)
    base = (c * _NS + s) * chunks * _KD

    def load(b, g):
        off = base + g * _KD
        pltpu.async_copy(row_hbm.at[pl.ds(off, _KD)], row_v[b], lsem[b])
        pltpu.async_copy(col_hbm.at[pl.ds(off, _KD)], col_v[b], lsem[b])

    load(0, 0)
    load(1, 1)

    def body(p, carry):
        for b in range(2):
            g = p * 2 + b

            @pl.when(g >= 2)
            def _wdrain():
                pltpu.make_async_copy(row_hbm.at[pl.ds(0, _KD)],
                                      rowp_v[b], wsem[b]).wait()

            pltpu.make_async_copy(row_hbm.at[pl.ds(0, _KD)],
                                  row_v[b], lsem[b]).wait()
            pltpu.make_async_copy(row_hbm.at[pl.ds(0, _KD)],
                                  col_v[b], lsem[b]).wait()
            for i in range(_KD // 16):
                sl = pl.ds(i * 16, 16)
                r = row_v[b][sl]
                q = col_v[b][sl]
                rowp_v[b][sl] = jnp.where(r == q, _N, r)
            off = base + g * _KD
            pltpu.async_copy(rowp_v[b], rowp_hbm.at[pl.ds(off, _KD)], wsem[b])
            pltpu.sync_copy(ones_v, acc.at[rowp_v[b]], add=True)

            @pl.when(g + 2 < chunks)
            def _refill():
                load(b, g + 2)
        return carry

    lax.fori_loop(0, chunks // 2, body, 0)
    for b in range(2):
        pltpu.make_async_copy(row_hbm.at[pl.ds(0, _KD)],
                              rowp_v[b], wsem[b]).wait()
    plsc.subcore_barrier()
    pltpu.sync_copy(acc.at[pl.ds(s * seg, seg)], seg_v)
    pltpu.sync_copy(seg_v, out_hbm.at[pl.ds(c * _NDP + s * seg, seg)])


def _spmm_body(mean_hbm, var_hbm, rowp_hbm, col_hbm, mo_hbm, vo_hbm,
               rp0, rp1, rp2, rp3, rp4, rp5, cl0, cl1, cl2, cl3, cl4, cl5,
               rows0, rows1, rows2, rows3, acc,
               g0, g1, g2, g3, i0, i1, i2, i3, i4, i5):
    c = lax.axis_index("c")
    s = lax.axis_index("s")
    rpt = _NRP // _NS
    rows = (rows0, rows1, rows2, rows3)
    rpc = (rp0, rp1, rp2, rp3, rp4, rp5)
    clc = (cl0, cl1, cl2, cl3, cl4, cl5)
    gsem = (g0, g1, g2, g3)
    isem = (i0, i1, i2, i3, i4, i5)
    base = s * _CPT * _K

    def fire_idx(q, g):
        off = base + g * _K
        pltpu.async_copy(rowp_hbm.at[pl.ds(off, _K)], rpc[q], isem[q])
        pltpu.async_copy(col_hbm.at[pl.ds(off, _K)], clc[q], isem[q])

    def wait_idx(q):
        pltpu.make_async_copy(rowp_hbm.at[pl.ds(0, _K)], rpc[q],
                              isem[q]).wait()
        pltpu.make_async_copy(rowp_hbm.at[pl.ds(0, _K)], clc[q],
                              isem[q]).wait()

    def run(src_hbm, out_hbm):
        pltpu.sync_copy(src_hbm.at[pl.ds(s * rpt, rpt)],
                        acc.at[pl.ds(s * rpt, rpt)])
        plsc.subcore_barrier()

        def fire_gather(r, q):
            pltpu.async_copy(src_hbm.at[clc[q]], rows[r], gsem[r])

        for q in range(_NIDX):
            fire_idx(q, q)
        for g in range(_NROWS - 1):
            wait_idx(g)
            fire_gather(g, g)

        def body(p, carry):
            for u in range(_UNROLL):
                g = p * _UNROLL + u
                r = u % _NROWS
                pltpu.make_async_copy(src_hbm.at[pl.ds(0, _K)],
                                      rows[r], gsem[r]).wait()
                pltpu.sync_copy(rows[r], acc.at[rpc[u % _NIDX]], add=True)

                @pl.when(g + _NIDX < _CPT)
                def _refill_idx():
                    fire_idx(u % _NIDX, g + _NIDX)

                @pl.when(g + 3 < _CPT)
                def _next_gather():
                    wait_idx((u + 3) % _NIDX)
                    fire_gather((u + 3) % _NROWS, (u + 3) % _NIDX)
            return carry

        lax.fori_loop(0, _CPT // _UNROLL, body, 0)
        plsc.subcore_barrier()
        pltpu.sync_copy(acc.at[pl.ds(s * rpt, rpt)],
                        out_hbm.at[pl.ds(s * rpt, rpt)])

    @pl.when(c == 0)
    def _mean():
        run(mean_hbm, mo_hbm)

    @pl.when(c == 1)
    def _var():
        run(var_hbm, vo_hbm)


_SC_CACHE = {}


def _deg_call(*args):
    if "deg" not in _SC_CACHE:
        _SC_CACHE["deg"] = pl.kernel(
            _deg_body,
            out_type=[jax.ShapeDtypeStruct((_NC * _NDP,), jnp.float32),
                      jax.ShapeDtypeStruct((_EPAD,), jnp.int32)],
            mesh=_sc_mesh(),
            scratch_types=(
                [pltpu.VMEM((_KD,), jnp.int32)] * 6
                + [pltpu.VMEM((_KD,), jnp.float32),
                   pltpu.VMEM((_NDP // _NS,), jnp.float32),
                   pltpu.VMEM_SHARED((_NDP,), jnp.float32)]
                + [pltpu.SemaphoreType.DMA] * 4
            ),
        )
    return _SC_CACHE["deg"](*args)


def _spmm_call(*args):
    if "spmm" not in _SC_CACHE:
        _SC_CACHE["spmm"] = pl.kernel(
            _spmm_body,
            out_type=[jax.ShapeDtypeStruct((_NRP, _F), jnp.float32)] * 2,
            mesh=_sc_mesh(),
            scratch_types=(
                [pltpu.VMEM((_K,), jnp.int32)] * (2 * _NIDX)
                + [pltpu.VMEM((_K, _F), jnp.float32)] * _NROWS
                + [pltpu.VMEM_SHARED((_NRP, _F), jnp.float32)]
                + [pltpu.SemaphoreType.DMA] * (_NROWS + _NIDX)
            ),
        )
    return _SC_CACHE["spmm"](*args)


def _elu(t):
    return jnp.where(t > 0, t, jnp.exp(t) - 1.0)


def _dot(a, b):
    return jnp.dot(a, b, preferred_element_type=jnp.float32)


def _scales(da, db):
    deg = da[...] + db[...] + 1.0
    return lax.rsqrt(deg), 1.0 / deg


def _tc1a_body(x, wm0, bm0, wv0, bv0, wm1, bm1, wv1, bv1, mo, vo):
    xb = x[...]
    m = _elu(_dot(xb, wm0[...]) + bm0[...])
    v = jnp.maximum(_dot(xb, wv0[...]) + bv0[...], 0.0)
    m = _elu(_dot(m, wm1[...]) + bm1[...])
    v = jnp.maximum(_dot(v, wv1[...]) + bv1[...], 0.0) + 1e-6
    att = jnp.exp(-v)
    mo[...] = m * att
    vo[...] = v * att * att


def _tc1b_body(m, v, da, db, mo, vo):
    d0, d1 = _scales(da, db)
    mo[...] = d0 * m[...]
    vo[...] = d1 * v[...]


def _tc2_body(ma, va, da, db, wm2, bm2, wv2, bv2, mo, vo):
    d0, d1 = _scales(da, db)
    m = d0 * ma[...]
    v = d1 * va[...]
    m = _elu(_dot(m, wm2[...]) + bm2[...])
    v = jnp.maximum(_dot(v, wv2[...]) + bv2[...], 0.0) + 1e-6
    att = jnp.exp(-v)
    mo[...] = d0 * (m * att)
    vo[...] = d1 * (v * att * att)


def _tc3_body(ma, va, da, db, smp, out):
    d0, d1 = _scales(da, db)
    m = d0 * ma[...]
    v = d1 * va[...]
    o = m + smp[...] * jnp.sqrt(v)
    o = o - jnp.max(o, axis=-1, keepdims=True)
    out[...] = o - jnp.log(jnp.sum(jnp.exp(o), axis=-1, keepdims=True))


def _row_spec():
    return pl.BlockSpec((_BLK, _F), lambda i: (i, 0))


def _deg_spec():
    return pl.BlockSpec((_BLK, 1), lambda i: (i, 0))


def _w_spec():
    return pl.BlockSpec((_F, _F), lambda i: (0, 0))


def _b_spec():
    return pl.BlockSpec((1, _F), lambda i: (0, 0))


_TC_PARAMS = pltpu.CompilerParams(dimension_semantics=("parallel",))

_PADDED_OUT = [jax.ShapeDtypeStruct((_NRP, _F), jnp.float32)] * 2

_tc1a = pl.pallas_call(
    _tc1a_body,
    grid=(_N // _BLK,),
    in_specs=[_row_spec(),
              _w_spec(), _b_spec(), _w_spec(), _b_spec(),
              _w_spec(), _b_spec(), _w_spec(), _b_spec()],
    out_specs=[_row_spec(), _row_spec()],
    out_shape=[jax.ShapeDtypeStruct((_N, _F), jnp.float32)] * 2,
    compiler_params=_TC_PARAMS,
)

_tc1b = pl.pallas_call(
    _tc1b_body,
    grid=(_N // _BLK,),
    in_specs=[_row_spec(), _row_spec(), _deg_spec(), _deg_spec()],
    out_specs=[_row_spec(), _row_spec()],
    out_shape=_PADDED_OUT,
    compiler_params=_TC_PARAMS,
)

_tc2 = pl.pallas_call(
    _tc2_body,
    grid=(_N // _BLK,),
    in_specs=[_row_spec(), _row_spec(), _deg_spec(), _deg_spec(),
              _w_spec(), _b_spec(), _w_spec(), _b_spec()],
    out_specs=[_row_spec(), _row_spec()],
    out_shape=_PADDED_OUT,
    compiler_params=_TC_PARAMS,
)

_tc3 = pl.pallas_call(
    _tc3_body,
    grid=(_N // _BLK,),
    in_specs=[_row_spec(), _row_spec(), _deg_spec(), _deg_spec(), _row_spec()],
    out_specs=_row_spec(),
    out_shape=jax.ShapeDtypeStruct((_N, _F), jnp.float32),
    compiler_params=_TC_PARAMS,
)


def kernel(x, edge_index, Wm0, bm0, Wv0, bv0, Wm1, bm1, Wv1, bv1,
           Wm2, bm2, Wv2, bv2):
    n, f = x.shape
    row = edge_index[0]
    col = edge_index[1]
    e = row.shape[0]
    zpi = jnp.zeros((_EPAD - e,), jnp.int32)
    row_p = jnp.concatenate([row, zpi])
    col_p = jnp.concatenate([col, zpi])

    degp, rowp = _deg_call(row_p, col_p)
    da = degp[:n].reshape(n, 1)
    db = degp[_NDP:_NDP + n].reshape(n, 1)

    rb = lambda t: t.reshape(1, -1)
    m1, v1 = _tc1a(x, Wm0, rb(bm0), Wv0, rb(bv0),
                   Wm1, rb(bm1), Wv1, rb(bv1))
    mean_s, var_s = _tc1b(m1, v1, da, db)

    ma, va = _spmm_call(mean_s, var_s, rowp, col_p)
    mean_s2, var_s2 = _tc2(ma, va, da, db, Wm2, rb(bm2), Wv2, rb(bv2))
    ma2, va2 = _spmm_call(mean_s2, var_s2, rowp, col_p)

    smp = jax.random.normal(jax.random.key(42), (n, f), jnp.float32)
    return _tc3(ma2, va2, da, db, smp)
